# baseline (speedup 1.0000x reference)
import jax
import jax.numpy as jnp
from jax.experimental import pallas as pl

N_NODES = 100000
D = 128
D_GRAPH = 256
NUM_GRAPHS = 1024

BLOCK = 5000
WIN = 64


def _fused_kernel(hv_ref, ids_ref, wg_ref, bg_ref, waug_ref, out_ref):
    step = pl.program_id(0)

    @pl.when(step == 0)
    def _init():
        out_ref[...] = jnp.zeros_like(out_ref)

    hv = hv_ref[...]
    ids = ids_ref[0, 0, :]

    gate_lin = jax.lax.dot_general(
        hv, wg_ref[...], (((1,), (0,)), ((), ())),
        preferred_element_type=jnp.float32)
    gate = 0.5 * jnp.tanh((gate_lin + bg_ref[0, 0]) * 0.5) + 0.5

    hv_aug = jnp.concatenate(
        [hv.astype(jnp.bfloat16),
         jnp.ones((BLOCK, 1), jnp.bfloat16)], axis=1)
    waug = waug_ref[...]

    first = jnp.min(ids)
    last = jnp.max(ids)
    w_base = (first // 8) * 8
    n_win = (last - w_base) // WIN + 1

    ids_col = ids.astype(jnp.int16)[:, None]
    lane = jax.lax.broadcasted_iota(jnp.int16, (BLOCK, WIN), 1)
    zero = jnp.zeros((BLOCK, WIN), jnp.bfloat16)
    gate16 = gate.astype(jnp.bfloat16)

    def body(k, _):
        w0 = w_base + k * WIN
        gated_onehot = jnp.where(
            ids_col == (lane + w0.astype(jnp.int16)), gate16, zero)
        seg_feats = jax.lax.dot_general(
            gated_onehot, hv_aug, (((0,), (0,)), ((), ())),
            preferred_element_type=jnp.float32)
        partial = jax.lax.dot_general(
            seg_feats, waug, (((1,), (0,)), ((), ())),
            preferred_element_type=jnp.float32)
        out_ref[pl.ds(w0, WIN), :] += partial
        return 0

    jax.lax.fori_loop(0, n_win, body, 0)


@jax.jit
def kernel(hv, segment_ids, W_gate, b_gate, W_g2g, b_g2g):
    ids = segment_ids.astype(jnp.int32).reshape(N_NODES // BLOCK, 1, BLOCK)
    bg = b_gate.reshape(1, 1)
    w_aug = jnp.concatenate([W_g2g, b_g2g.reshape(1, D_GRAPH)], axis=0)
    grid = (N_NODES // BLOCK,)
    out_padded = pl.pallas_call(
        _fused_kernel,
        grid=grid,
        in_specs=[
            pl.BlockSpec((BLOCK, D), lambda i: (i, 0)),
            pl.BlockSpec((1, 1, BLOCK), lambda i: (i, 0, 0)),
            pl.BlockSpec((D, 1), lambda i: (0, 0)),
            pl.BlockSpec((1, 1), lambda i: (0, 0)),
            pl.BlockSpec((D + 1, D_GRAPH), lambda i: (0, 0)),
        ],
        out_specs=pl.BlockSpec((NUM_GRAPHS + WIN, D_GRAPH), lambda i: (0, 0)),
        out_shape=jax.ShapeDtypeStruct((NUM_GRAPHS + WIN, D_GRAPH), jnp.float32),
    )(hv, ids, W_gate, bg, w_aug)
    return out_padded[:NUM_GRAPHS]


# speedup vs baseline: 1.0317x; 1.0317x over previous
import jax
import jax.numpy as jnp
from jax.experimental import pallas as pl

N_NODES = 100000
D = 128
D_GRAPH = 256
NUM_GRAPHS = 1024

BLOCK = 20000
WIN = 232


def _fused_kernel(hv_ref, ids_ref, wg_ref, bg_ref, waug_ref, out_ref):
    step = pl.program_id(0)

    @pl.when(step == 0)
    def _init():
        out_ref[...] = jnp.zeros_like(out_ref)

    hv = hv_ref[...]
    ids = ids_ref[0, 0, :]

    gate_lin = jax.lax.dot_general(
        hv, wg_ref[...], (((1,), (0,)), ((), ())),
        preferred_element_type=jnp.float32)
    gate = 0.5 * jnp.tanh((gate_lin + bg_ref[0, 0]) * 0.5) + 0.5

    hv_aug = jnp.concatenate(
        [hv.astype(jnp.bfloat16),
         jnp.ones((BLOCK, 1), jnp.bfloat16)], axis=1)
    waug = waug_ref[...]

    first = jnp.min(ids)
    last = jnp.max(ids)
    w_base = (first // 8) * 8
    n_win = (last - w_base) // WIN + 1

    ids_col = ids.astype(jnp.int16)[:, None]
    lane = jax.lax.broadcasted_iota(jnp.int16, (BLOCK, WIN), 1)
    zero = jnp.zeros((BLOCK, WIN), jnp.bfloat16)
    gate16 = gate.astype(jnp.bfloat16)

    def body(k, _):
        w0 = w_base + k * WIN
        gated_onehot = jnp.where(
            ids_col == (lane + w0.astype(jnp.int16)), gate16, zero)
        seg_feats = jax.lax.dot_general(
            gated_onehot, hv_aug, (((0,), (0,)), ((), ())),
            preferred_element_type=jnp.float32)
        partial = jax.lax.dot_general(
            seg_feats, waug, (((1,), (0,)), ((), ())),
            preferred_element_type=jnp.float32)
        out_ref[pl.ds(w0, WIN), :] += partial
        return 0

    jax.lax.fori_loop(0, n_win, body, 0)


@jax.jit
def kernel(hv, segment_ids, W_gate, b_gate, W_g2g, b_g2g):
    ids = segment_ids.astype(jnp.int32).reshape(N_NODES // BLOCK, 1, BLOCK)
    bg = b_gate.reshape(1, 1)
    w_aug = jnp.concatenate([W_g2g, b_g2g.reshape(1, D_GRAPH)], axis=0)
    grid = (N_NODES // BLOCK,)
    out_padded = pl.pallas_call(
        _fused_kernel,
        grid=grid,
        in_specs=[
            pl.BlockSpec((BLOCK, D), lambda i: (i, 0)),
            pl.BlockSpec((1, 1, BLOCK), lambda i: (i, 0, 0)),
            pl.BlockSpec((D, 1), lambda i: (0, 0)),
            pl.BlockSpec((1, 1), lambda i: (0, 0)),
            pl.BlockSpec((D + 1, D_GRAPH), lambda i: (0, 0)),
        ],
        out_specs=pl.BlockSpec((NUM_GRAPHS + WIN, D_GRAPH), lambda i: (0, 0)),
        out_shape=jax.ShapeDtypeStruct((NUM_GRAPHS + WIN, D_GRAPH), jnp.float32),
    )(hv, ids, W_gate, bg, w_aug)
    return out_padded[:NUM_GRAPHS]


# speedup vs baseline: 1.1451x; 1.1100x over previous
import jax
import jax.numpy as jnp
from jax.experimental import pallas as pl

N_NODES = 100000
D = 128
D_GRAPH = 256
NUM_GRAPHS = 1024

BLOCK = 10000
WIN = 112


def _fused_kernel(hv_ref, ids_ref, wg_ref, bg_ref, waug_ref, out_ref):
    step = pl.program_id(0)

    @pl.when(step == 0)
    def _init():
        out_ref[...] = jnp.zeros_like(out_ref)

    hv = hv_ref[...]
    ids = ids_ref[0, 0, :]

    gate_lin = jax.lax.dot_general(
        hv, wg_ref[...], (((1,), (0,)), ((), ())),
        preferred_element_type=jnp.float32)
    gate = 0.5 * jnp.tanh((gate_lin + bg_ref[0, 0]) * 0.5) + 0.5

    hv_aug = jnp.concatenate(
        [hv.astype(jnp.bfloat16),
         jnp.ones((BLOCK, 1), jnp.bfloat16)], axis=1)
    waug = waug_ref[...]

    first = jnp.min(ids)
    last = jnp.max(ids)
    w_base = (first // 8) * 8
    n_win = (last - w_base) // WIN + 1

    ids_col = ids.astype(jnp.int16)[:, None]
    lane = jax.lax.broadcasted_iota(jnp.int16, (BLOCK, WIN), 1)
    zero = jnp.zeros((BLOCK, WIN), jnp.bfloat16)
    gate16 = gate.astype(jnp.bfloat16)

    def body(k, _):
        w0 = w_base + k * WIN
        gated_onehot = jnp.where(
            ids_col == (lane + w0.astype(jnp.int16)), gate16, zero)
        seg_feats = jax.lax.dot_general(
            gated_onehot, hv_aug, (((0,), (0,)), ((), ())),
            preferred_element_type=jnp.float32)
        partial = jax.lax.dot_general(
            seg_feats, waug, (((1,), (0,)), ((), ())),
            preferred_element_type=jnp.float32)
        out_ref[pl.ds(w0, WIN), :] += partial
        return 0

    jax.lax.fori_loop(0, n_win, body, 0)


@jax.jit
def kernel(hv, segment_ids, W_gate, b_gate, W_g2g, b_g2g):
    ids = segment_ids.astype(jnp.int32).reshape(N_NODES // BLOCK, 1, BLOCK)
    bg = b_gate.reshape(1, 1)
    w_aug = jnp.concatenate([W_g2g, b_g2g.reshape(1, D_GRAPH)], axis=0)
    grid = (N_NODES // BLOCK,)
    out_padded = pl.pallas_call(
        _fused_kernel,
        grid=grid,
        in_specs=[
            pl.BlockSpec((BLOCK, D), lambda i: (i, 0)),
            pl.BlockSpec((1, 1, BLOCK), lambda i: (i, 0, 0)),
            pl.BlockSpec((D, 1), lambda i: (0, 0)),
            pl.BlockSpec((1, 1), lambda i: (0, 0)),
            pl.BlockSpec((D + 1, D_GRAPH), lambda i: (0, 0)),
        ],
        out_specs=pl.BlockSpec((NUM_GRAPHS + WIN, D_GRAPH), lambda i: (0, 0)),
        out_shape=jax.ShapeDtypeStruct((NUM_GRAPHS + WIN, D_GRAPH), jnp.float32),
    )(hv, ids, W_gate, bg, w_aug)
    return out_padded[:NUM_GRAPHS]
